# baseline (speedup 1.0000x reference)
import functools

import jax
import jax.numpy as jnp
from jax import lax
from jax.experimental import pallas as pl
from jax.experimental.pallas import tpu as pltpu
from jax.experimental.pallas import tpu_sc as plsc

NS = 16
NC = 2
K = 128
HALF = 128


def _make_seg_sum(n, e):
    eps = e // NS
    n_chunk = eps // K
    assert n_chunk % 2 == 1, "pipeline below assumes an odd chunk count"
    zr = K
    n_pad = -(-(n + 1) // (NS * zr)) * NS * zr
    rps = n_pad // NS
    n_wb = rps // zr

    mesh = plsc.VectorSubcoreMesh(core_axis_name="c", subcore_axis_name="s")

    @functools.partial(
        pl.kernel,
        out_type=jax.ShapeDtypeStruct((NC, n_pad, HALF), jnp.float32),
        mesh=mesh,
        scratch_types=[
            pltpu.VMEM((n_chunk, K), jnp.int32),
            pltpu.VMEM((K,), jnp.int32),
            pltpu.VMEM((K,), jnp.int32),
            pltpu.VMEM((K, HALF), jnp.float32),
            pltpu.VMEM((K, HALF), jnp.float32),
            pltpu.VMEM_SHARED((n_pad, HALF), jnp.float32),
            pltpu.SemaphoreType.DMA,
            pltpu.SemaphoreType.DMA,
            pltpu.SemaphoreType.DMA,
            pltpu.SemaphoreType.DMA,
        ],
    )
    def seg_sum(table_hbm, src_hbm, dst_hbm, out_hbm,
                src_all, dst_a, dst_b, rows_a, rows_b, acc_sh,
                sem_a, sem_b, sem_da, sem_db):
        c = lax.axis_index("c")
        s = lax.axis_index("s")

        def zero_row(i, _):
            for jj in range(HALF // 16):
                rows_a[i, pl.ds(jj * 16, 16)] = jnp.zeros((16,), jnp.float32)
            return 0
        lax.fori_loop(0, zr, zero_row, 0)
        for r in range(n_wb):
            pltpu.async_copy(rows_a, acc_sh.at[pl.ds(s * rps + r * zr, zr)], sem_a)

        pltpu.sync_copy(src_hbm.at[s], src_all)
        pltpu.sync_copy(dst_hbm.at[s, 0], dst_a)
        for r in range(n_wb):
            pltpu.make_async_copy(rows_a, acc_sh.at[pl.ds(s * rps + r * zr, zr)],
                                  sem_a).wait()

        def gather_start(k, buf, sem):
            return pltpu.async_copy(table_hbm.at[c].at[src_all.at[k]], buf, sem)

        def gather_wait(k, buf, sem):
            pltpu.make_async_copy(table_hbm.at[c].at[src_all.at[k]], buf, sem).wait()

        def dst_start(k, buf, sem):
            return pltpu.async_copy(dst_hbm.at[s, k], buf, sem)

        def dst_wait(k, buf, sem):
            pltpu.make_async_copy(dst_hbm.at[s, k], buf, sem).wait()

        def scatter(k, dbuf, buf):
            pltpu.sync_copy(buf, acc_sh.at[dbuf], add=True)

        gather_start(0, rows_a, sem_a)
        dst_start(1, dst_b, sem_db)
        plsc.subcore_barrier()

        def body(p, _):
            k = 2 * p
            gather_wait(k, rows_a, sem_a)
            gather_start(k + 1, rows_b, sem_b)
            scatter(k, dst_a, rows_a)
            dst_start(k + 2, dst_a, sem_da)
            gather_wait(k + 1, rows_b, sem_b)
            gather_start(k + 2, rows_a, sem_a)
            dst_wait(k + 1, dst_b, sem_db)
            scatter(k + 1, dst_b, rows_b)
            dst_start(k + 3, dst_b, sem_db)
            dst_wait(k + 2, dst_a, sem_da)
            return 0
        lax.fori_loop(0, (n_chunk - 1) // 2, body, 0)
        gather_wait(n_chunk - 1, rows_a, sem_a)
        scatter(n_chunk - 1, dst_a, rows_a)
        dst_wait(n_chunk, dst_b, sem_db)
        plsc.subcore_barrier()

        bufs = (rows_a, rows_b)
        sems = (sem_a, sem_b)
        pltpu.async_copy(acc_sh.at[pl.ds(s * rps, zr)], rows_a, sem_a)
        for r in range(n_wb):
            row = s * rps + r * zr
            pltpu.make_async_copy(acc_sh.at[pl.ds(row, zr)], bufs[r % 2],
                                  sems[r % 2]).wait()
            if r + 1 < n_wb:
                pltpu.async_copy(acc_sh.at[pl.ds(row + zr, zr)],
                                 bufs[(r + 1) % 2], sems[(r + 1) % 2])
            pltpu.sync_copy(bufs[r % 2], out_hbm.at[c, pl.ds(row, zr)])

    return seg_sum


def _layer1_body(agg_ref, x_ref, wrel_ref, wroot_ref, b_ref, out_ref):
    a0 = agg_ref[0]
    a1 = agg_ref[1]
    wrel = wrel_ref[...]
    h = jnp.dot(a0, wrel[:HALF, :], preferred_element_type=jnp.float32)
    h += jnp.dot(a1, wrel[HALF:, :], preferred_element_type=jnp.float32)
    h += jnp.dot(x_ref[...], wroot_ref[...], preferred_element_type=jnp.float32)
    h += b_ref[...]
    t = jnp.maximum(h, 0.0)
    out_ref[0] = t[:, :HALF]
    out_ref[1] = t[:, HALF:]


def _layer2_body(agg_ref, t_ref, wrel_ref, wroot_ref, b_ref, out_ref):
    wrel = wrel_ref[...]
    wroot = wroot_ref[...]
    h = jnp.dot(agg_ref[0], wrel[:HALF, :], preferred_element_type=jnp.float32)
    h += jnp.dot(agg_ref[1], wrel[HALF:, :], preferred_element_type=jnp.float32)
    h += jnp.dot(t_ref[0], wroot[:HALF, :], preferred_element_type=jnp.float32)
    h += jnp.dot(t_ref[1], wroot[HALF:, :], preferred_element_type=jnp.float32)
    h += b_ref[...]
    out_ref[...] = h


def _tc_layer1(agg, x, wrel, wroot, b, bn):
    n, d = x.shape
    grid = (n // bn,)
    return pl.pallas_call(
        _layer1_body,
        grid=grid,
        in_specs=[
            pl.BlockSpec((2, bn, HALF), lambda i: (0, i, 0)),
            pl.BlockSpec((bn, d), lambda i: (i, 0)),
            pl.BlockSpec((d, d), lambda i: (0, 0)),
            pl.BlockSpec((d, d), lambda i: (0, 0)),
            pl.BlockSpec((1, d), lambda i: (0, 0)),
        ],
        out_specs=pl.BlockSpec((2, bn, HALF), lambda i: (0, i, 0)),
        out_shape=jax.ShapeDtypeStruct((2, n, HALF), jnp.float32),
    )(agg, x, wrel, wroot, b.reshape(1, d))


def _tc_layer2(agg, t_split, wrel, wroot, b, bn):
    n = agg.shape[1]
    d = 2 * HALF
    grid = (n // bn,)
    return pl.pallas_call(
        _layer2_body,
        grid=grid,
        in_specs=[
            pl.BlockSpec((2, bn, HALF), lambda i: (0, i, 0)),
            pl.BlockSpec((2, bn, HALF), lambda i: (0, i, 0)),
            pl.BlockSpec((d, d), lambda i: (0, 0)),
            pl.BlockSpec((d, d), lambda i: (0, 0)),
            pl.BlockSpec((1, d), lambda i: (0, 0)),
        ],
        out_specs=pl.BlockSpec((bn, d), lambda i: (i, 0)),
        out_shape=jax.ShapeDtypeStruct((n, d), jnp.float32),
    )(agg, t_split, wrel, wroot, b.reshape(1, d))


def kernel(x, edge_index, W1_rel, W1_root, b1, W2_rel, W2_root, b2):
    n, d = x.shape
    e = edge_index.shape[1]
    n_chunk = -(-e // (NS * K))
    if n_chunk % 2 == 0:
        n_chunk += 1
    e_pad = NS * K * n_chunk
    src = jnp.pad(edge_index[0], (0, e_pad - e)).reshape(NS, n_chunk, K)
    dst_flat = jnp.pad(edge_index[1], (0, e_pad - e), constant_values=n)
    dst = jnp.pad(dst_flat.reshape(NS, n_chunk, K), ((0, 0), (0, 1), (0, 0)))
    bn = 1000

    seg_sum = _make_seg_sum(n, e_pad)

    x2 = x.reshape(n, 2, HALF).transpose(1, 0, 2)
    agg1 = seg_sum(x2, src, dst)[:, :n]
    t_split = _tc_layer1(agg1, x, W1_rel, W1_root, b1, bn)

    agg2 = seg_sum(t_split, src, dst)[:, :n]
    return _tc_layer2(agg2, t_split, W2_rel, W2_root, b2, bn)


# speedup vs baseline: 1.2812x; 1.2812x over previous
import functools

import jax
import jax.numpy as jnp
from jax import lax
from jax.experimental import pallas as pl
from jax.experimental.pallas import tpu as pltpu
from jax.experimental.pallas import tpu_sc as plsc

NS = 16
NC = 2
K = 80
HALF = 128


def _make_seg_sum(n, e):
    eps = e // NS
    n_chunk = eps // K
    assert n_chunk % 2 == 1, "pipeline below assumes an odd chunk count"
    zr = K
    n_pad = -(-(n + 1) // (NS * zr)) * NS * zr
    rps = n_pad // NS
    n_wb = rps // zr

    mesh = plsc.VectorSubcoreMesh(core_axis_name="c", subcore_axis_name="s")

    @functools.partial(
        pl.kernel,
        out_type=jax.ShapeDtypeStruct((NC, n_pad, HALF), jnp.float32),
        mesh=mesh,
        scratch_types=[
            pltpu.VMEM((n_chunk, K), jnp.int32),
            pltpu.VMEM((K,), jnp.int32),
            pltpu.VMEM((K,), jnp.int32),
            pltpu.VMEM((K, HALF), jnp.float32),
            pltpu.VMEM((K, HALF), jnp.float32),
            pltpu.VMEM_SHARED((n_pad, HALF), jnp.float32),
            pltpu.SemaphoreType.DMA,
            pltpu.SemaphoreType.DMA,
            pltpu.SemaphoreType.DMA,
            pltpu.SemaphoreType.DMA,
        ],
    )
    def seg_sum(table_hbm, src_hbm, dst_hbm, out_hbm,
                src_all, dst_a, dst_b, rows_a, rows_b, acc_sh,
                sem_a, sem_b, sem_da, sem_db):
        c = lax.axis_index("c")
        s = lax.axis_index("s")

        def zero_row(i, _):
            for jj in range(HALF // 16):
                rows_a[i, pl.ds(jj * 16, 16)] = jnp.zeros((16,), jnp.float32)
            return 0
        lax.fori_loop(0, zr, zero_row, 0)
        for r in range(n_wb):
            pltpu.async_copy(rows_a, acc_sh.at[pl.ds(s * rps + r * zr, zr)], sem_a)

        pltpu.sync_copy(src_hbm.at[s], src_all)
        pltpu.sync_copy(dst_hbm.at[s, 0], dst_a)
        for r in range(n_wb):
            pltpu.make_async_copy(rows_a, acc_sh.at[pl.ds(s * rps + r * zr, zr)],
                                  sem_a).wait()

        def gather_start(k, buf, sem):
            return pltpu.async_copy(table_hbm.at[c].at[src_all.at[k]], buf, sem)

        def gather_wait(k, buf, sem):
            pltpu.make_async_copy(table_hbm.at[c].at[src_all.at[k]], buf, sem).wait()

        def dst_start(k, buf, sem):
            return pltpu.async_copy(dst_hbm.at[s, k], buf, sem)

        def dst_wait(k, buf, sem):
            pltpu.make_async_copy(dst_hbm.at[s, k], buf, sem).wait()

        def scatter(k, dbuf, buf):
            pltpu.sync_copy(buf, acc_sh.at[dbuf], add=True)

        gather_start(0, rows_a, sem_a)
        dst_start(1, dst_b, sem_db)
        plsc.subcore_barrier()

        def body(p, _):
            k = 2 * p
            gather_wait(k, rows_a, sem_a)
            gather_start(k + 1, rows_b, sem_b)
            scatter(k, dst_a, rows_a)
            dst_start(k + 2, dst_a, sem_da)
            gather_wait(k + 1, rows_b, sem_b)
            gather_start(k + 2, rows_a, sem_a)
            dst_wait(k + 1, dst_b, sem_db)
            scatter(k + 1, dst_b, rows_b)
            dst_start(k + 3, dst_b, sem_db)
            dst_wait(k + 2, dst_a, sem_da)
            return 0
        lax.fori_loop(0, (n_chunk - 1) // 2, body, 0)
        gather_wait(n_chunk - 1, rows_a, sem_a)
        scatter(n_chunk - 1, dst_a, rows_a)
        dst_wait(n_chunk, dst_b, sem_db)
        plsc.subcore_barrier()

        bufs = (rows_a, rows_b)
        sems = (sem_a, sem_b)
        pltpu.async_copy(acc_sh.at[pl.ds(s * rps, zr)], rows_a, sem_a)
        for r in range(n_wb):
            row = s * rps + r * zr
            pltpu.make_async_copy(acc_sh.at[pl.ds(row, zr)], bufs[r % 2],
                                  sems[r % 2]).wait()
            if r + 1 < n_wb:
                pltpu.async_copy(acc_sh.at[pl.ds(row + zr, zr)],
                                 bufs[(r + 1) % 2], sems[(r + 1) % 2])
            pltpu.sync_copy(bufs[r % 2], out_hbm.at[c, pl.ds(row, zr)])

    return seg_sum


def _layer1_body(agg_ref, x_ref, wrel_ref, wroot_ref, b_ref, out_ref):
    a0 = agg_ref[0]
    a1 = agg_ref[1]
    wrel = wrel_ref[...]
    h = jnp.dot(a0, wrel[:HALF, :], preferred_element_type=jnp.float32)
    h += jnp.dot(a1, wrel[HALF:, :], preferred_element_type=jnp.float32)
    h += jnp.dot(x_ref[...], wroot_ref[...], preferred_element_type=jnp.float32)
    h += b_ref[...]
    t = jnp.maximum(h, 0.0)
    out_ref[0] = t[:, :HALF]
    out_ref[1] = t[:, HALF:]


def _layer2_body(agg_ref, t_ref, wrel_ref, wroot_ref, b_ref, out_ref):
    wrel = wrel_ref[...]
    wroot = wroot_ref[...]
    h = jnp.dot(agg_ref[0], wrel[:HALF, :], preferred_element_type=jnp.float32)
    h += jnp.dot(agg_ref[1], wrel[HALF:, :], preferred_element_type=jnp.float32)
    h += jnp.dot(t_ref[0], wroot[:HALF, :], preferred_element_type=jnp.float32)
    h += jnp.dot(t_ref[1], wroot[HALF:, :], preferred_element_type=jnp.float32)
    h += b_ref[...]
    out_ref[...] = h


def _tc_layer1(agg, x, wrel, wroot, b, bn):
    n, d = x.shape
    grid = (n // bn,)
    return pl.pallas_call(
        _layer1_body,
        grid=grid,
        in_specs=[
            pl.BlockSpec((2, bn, HALF), lambda i: (0, i, 0)),
            pl.BlockSpec((bn, d), lambda i: (i, 0)),
            pl.BlockSpec((d, d), lambda i: (0, 0)),
            pl.BlockSpec((d, d), lambda i: (0, 0)),
            pl.BlockSpec((1, d), lambda i: (0, 0)),
        ],
        out_specs=pl.BlockSpec((2, bn, HALF), lambda i: (0, i, 0)),
        out_shape=jax.ShapeDtypeStruct((2, n, HALF), jnp.float32),
    )(agg, x, wrel, wroot, b.reshape(1, d))


def _tc_layer2(agg, t_split, wrel, wroot, b, bn):
    n = agg.shape[1]
    d = 2 * HALF
    grid = (n // bn,)
    return pl.pallas_call(
        _layer2_body,
        grid=grid,
        in_specs=[
            pl.BlockSpec((2, bn, HALF), lambda i: (0, i, 0)),
            pl.BlockSpec((2, bn, HALF), lambda i: (0, i, 0)),
            pl.BlockSpec((d, d), lambda i: (0, 0)),
            pl.BlockSpec((d, d), lambda i: (0, 0)),
            pl.BlockSpec((1, d), lambda i: (0, 0)),
        ],
        out_specs=pl.BlockSpec((bn, d), lambda i: (i, 0)),
        out_shape=jax.ShapeDtypeStruct((n, d), jnp.float32),
    )(agg, t_split, wrel, wroot, b.reshape(1, d))


def kernel(x, edge_index, W1_rel, W1_root, b1, W2_rel, W2_root, b2):
    n, d = x.shape
    e = edge_index.shape[1]
    n_chunk = -(-e // (NS * K))
    if n_chunk % 2 == 0:
        n_chunk += 1
    e_pad = NS * K * n_chunk
    src = jnp.pad(edge_index[0], (0, e_pad - e)).reshape(NS, n_chunk, K)
    dst_flat = jnp.pad(edge_index[1], (0, e_pad - e), constant_values=n)
    dst = jnp.pad(dst_flat.reshape(NS, n_chunk, K), ((0, 0), (0, 1), (0, 0)))
    bn = 1000

    seg_sum = _make_seg_sum(n, e_pad)

    x2 = x.reshape(n, 2, HALF).transpose(1, 0, 2)
    agg1 = seg_sum(x2, src, dst)[:, :n]
    t_split = _tc_layer1(agg1, x, W1_rel, W1_root, b1, bn)

    agg2 = seg_sum(t_split, src, dst)[:, :n]
    return _tc_layer2(agg2, t_split, W2_rel, W2_root, b2, bn)


# speedup vs baseline: 1.2896x; 1.0066x over previous
import functools

import jax
import jax.numpy as jnp
from jax import lax
from jax.experimental import pallas as pl
from jax.experimental.pallas import tpu as pltpu
from jax.experimental.pallas import tpu_sc as plsc

NS = 16
NC = 2
K = 80
HALF = 128


def _make_seg_sum(n, e, mult, coeff):
    eps = e // NS
    n_chunk = eps // K
    assert n_chunk % 2 == 1, "pipeline below assumes an odd chunk count"
    zr = K
    n_pad = -(-(n + 1) // (NS * zr)) * NS * zr
    rps = n_pad // NS
    n_wb = rps // zr

    mesh = plsc.VectorSubcoreMesh(core_axis_name="c", subcore_axis_name="s")

    @functools.partial(
        pl.kernel,
        out_type=jax.ShapeDtypeStruct((NC, n_pad, HALF), jnp.float32),
        mesh=mesh,
        scratch_types=[
            pltpu.VMEM((n_chunk, K), jnp.int32),
            pltpu.VMEM((K,), jnp.int32),
            pltpu.VMEM((K,), jnp.int32),
            pltpu.VMEM((K, HALF), jnp.float32),
            pltpu.VMEM((K, HALF), jnp.float32),
            pltpu.VMEM_SHARED((n_pad, HALF), jnp.float32),
            pltpu.SemaphoreType.DMA,
            pltpu.SemaphoreType.DMA,
            pltpu.SemaphoreType.DMA,
            pltpu.SemaphoreType.DMA,
        ],
    )
    def seg_sum(table_hbm, src_hbm, dst_hbm, out_hbm,
                src_all, dst_a, dst_b, rows_a, rows_b, acc_sh,
                sem_a, sem_b, sem_da, sem_db):
        c = lax.axis_index("c")
        s = lax.axis_index("s")

        def zero_row(i, _):
            for jj in range(HALF // 16):
                rows_a[i, pl.ds(jj * 16, 16)] = jnp.zeros((16,), jnp.float32)
            return 0
        lax.fori_loop(0, zr, zero_row, 0)
        for r in range(n_wb):
            pltpu.async_copy(rows_a, acc_sh.at[pl.ds(s * rps + r * zr, zr)], sem_a)

        pltpu.sync_copy(src_hbm.at[s], src_all)
        pltpu.sync_copy(dst_hbm.at[s, 0], dst_a)
        pltpu.async_copy(dst_hbm.at[s, min(1, n_chunk - 1)], dst_b, sem_db)

        add = c * coeff
        def idx_row(i, _):
            for j in range(K // 16):
                v = src_all[i, pl.ds(j * 16, 16)]
                src_all[i, pl.ds(j * 16, 16)] = v * mult + add
            return 0
        lax.fori_loop(0, n_chunk, idx_row, 0)

        for r in range(n_wb):
            pltpu.make_async_copy(rows_a, acc_sh.at[pl.ds(s * rps + r * zr, zr)],
                                  sem_a).wait()

        def gather_start(k, buf, sem):
            return pltpu.async_copy(table_hbm.at[src_all.at[k]], buf, sem)

        def gather_wait(k, buf, sem):
            pltpu.make_async_copy(table_hbm.at[src_all.at[k]], buf, sem).wait()

        def dst_start(k, buf, sem):
            return pltpu.async_copy(dst_hbm.at[s, k], buf, sem)

        def dst_wait(k, buf, sem):
            pltpu.make_async_copy(dst_hbm.at[s, k], buf, sem).wait()

        def scatter(dbuf, buf):
            pltpu.sync_copy(buf, acc_sh.at[dbuf], add=True)

        gather_start(0, rows_a, sem_a)
        plsc.subcore_barrier()

        last = n_chunk - 1
        def body(p, _):
            k = 2 * p
            gather_wait(k, rows_a, sem_a)
            gather_start(k + 1, rows_b, sem_b)
            scatter(dst_a, rows_a)
            dst_start(k + 2, dst_a, sem_da)
            gather_wait(k + 1, rows_b, sem_b)
            gather_start(k + 2, rows_a, sem_a)
            dst_wait(k + 1, dst_b, sem_db)
            scatter(dst_b, rows_b)
            dst_start(jnp.minimum(k + 3, last), dst_b, sem_db)
            dst_wait(k + 2, dst_a, sem_da)
            return 0
        lax.fori_loop(0, (n_chunk - 1) // 2, body, 0)
        gather_wait(last, rows_a, sem_a)
        scatter(dst_a, rows_a)
        if n_chunk > 1:
            dst_wait(0, dst_b, sem_db)
        plsc.subcore_barrier()

        bufs = (rows_a, rows_b)
        sems = (sem_a, sem_b)
        pltpu.async_copy(acc_sh.at[pl.ds(s * rps, zr)], rows_a, sem_a)
        for r in range(n_wb):
            row = s * rps + r * zr
            pltpu.make_async_copy(acc_sh.at[pl.ds(row, zr)], bufs[r % 2],
                                  sems[r % 2]).wait()
            if r + 1 < n_wb:
                pltpu.async_copy(acc_sh.at[pl.ds(row + zr, zr)],
                                 bufs[(r + 1) % 2], sems[(r + 1) % 2])
            pltpu.sync_copy(bufs[r % 2], out_hbm.at[c, pl.ds(row, zr)])

    return seg_sum


def _pre_body(x_ref, w_ref, b_ref, out_ref):
    out_ref[...] = (
        jnp.dot(x_ref[...], w_ref[...], preferred_element_type=jnp.float32)
        + b_ref[...]
    )


def _pre_split_body(t_ref, w_ref, b_ref, out_ref):
    w = w_ref[...]
    out_ref[...] = (
        jnp.dot(t_ref[0], w[:HALF, :], preferred_element_type=jnp.float32)
        + jnp.dot(t_ref[1], w[HALF:, :], preferred_element_type=jnp.float32)
        + b_ref[...]
    )


def _combine_relu_body(agg_ref, pre_ref, wrel_ref, out_ref):
    wrel = wrel_ref[...]
    h = jnp.dot(agg_ref[0], wrel[:HALF, :], preferred_element_type=jnp.float32)
    h += jnp.dot(agg_ref[1], wrel[HALF:, :], preferred_element_type=jnp.float32)
    h += pre_ref[...]
    t = jnp.maximum(h, 0.0)
    out_ref[0] = t[:, :HALF]
    out_ref[1] = t[:, HALF:]


def _combine_body(agg_ref, pre_ref, wrel_ref, out_ref):
    wrel = wrel_ref[...]
    h = jnp.dot(agg_ref[0], wrel[:HALF, :], preferred_element_type=jnp.float32)
    h += jnp.dot(agg_ref[1], wrel[HALF:, :], preferred_element_type=jnp.float32)
    h += pre_ref[...]
    out_ref[...] = h


def _tc_pre(x, w, b, bn):
    n, d = x.shape
    return pl.pallas_call(
        _pre_body,
        grid=(n // bn,),
        in_specs=[
            pl.BlockSpec((bn, d), lambda i: (i, 0)),
            pl.BlockSpec((d, d), lambda i: (0, 0)),
            pl.BlockSpec((1, d), lambda i: (0, 0)),
        ],
        out_specs=pl.BlockSpec((bn, d), lambda i: (i, 0)),
        out_shape=jax.ShapeDtypeStruct((n, d), jnp.float32),
    )(x, w, b.reshape(1, d))


def _tc_pre_split(t_split, w, b, bn):
    n = t_split.shape[1]
    d = 2 * HALF
    return pl.pallas_call(
        _pre_split_body,
        grid=(n // bn,),
        in_specs=[
            pl.BlockSpec((2, bn, HALF), lambda i: (0, i, 0)),
            pl.BlockSpec((d, d), lambda i: (0, 0)),
            pl.BlockSpec((1, d), lambda i: (0, 0)),
        ],
        out_specs=pl.BlockSpec((bn, d), lambda i: (i, 0)),
        out_shape=jax.ShapeDtypeStruct((n, d), jnp.float32),
    )(t_split, w, b.reshape(1, d))


def _tc_combine(agg, pre, wrel, bn, relu):
    n, d = pre.shape
    body = _combine_relu_body if relu else _combine_body
    if relu:
        out_shape = jax.ShapeDtypeStruct((2, n, HALF), jnp.float32)
        out_specs = pl.BlockSpec((2, bn, HALF), lambda i: (0, i, 0))
    else:
        out_shape = jax.ShapeDtypeStruct((n, d), jnp.float32)
        out_specs = pl.BlockSpec((bn, d), lambda i: (i, 0))
    return pl.pallas_call(
        body,
        grid=(n // bn,),
        in_specs=[
            pl.BlockSpec((2, bn, HALF), lambda i: (0, i, 0)),
            pl.BlockSpec((bn, d), lambda i: (i, 0)),
            pl.BlockSpec((d, d), lambda i: (0, 0)),
        ],
        out_specs=out_specs,
        out_shape=out_shape,
    )(agg, pre, wrel)


def kernel(x, edge_index, W1_rel, W1_root, b1, W2_rel, W2_root, b2):
    n, d = x.shape
    e = edge_index.shape[1]
    n_chunk = e // (NS * K)
    src = edge_index[0].reshape(NS, n_chunk, K)
    dst = edge_index[1].reshape(NS, n_chunk, K)
    bn = 1000

    x2 = x.reshape(2 * n, HALF)
    pre1 = _tc_pre(x, W1_root, b1, bn)
    agg1 = _make_seg_sum(n, e, 2, 1)(x2, src, dst)[:, :n]
    t_split = _tc_combine(agg1, pre1, W1_rel, bn, relu=True)

    t2 = t_split.reshape(2 * n, HALF)
    pre2 = _tc_pre_split(t_split, W2_root, b2, bn)
    agg2 = _make_seg_sum(n, e, 1, n)(t2, src, dst)[:, :n]
    return _tc_combine(agg2, pre2, W2_rel, bn, relu=False)


# speedup vs baseline: 1.2945x; 1.0038x over previous
import functools

import jax
import jax.numpy as jnp
from jax import lax
from jax.experimental import pallas as pl
from jax.experimental.pallas import tpu as pltpu
from jax.experimental.pallas import tpu_sc as plsc

NS = 16
NC = 2
K = 80
HALF = 128


def _make_seg_sum(n, e, mult, coeff):
    eps = e // NS
    n_chunk = eps // K
    assert n_chunk % 2 == 1, "pipeline below assumes an odd chunk count"
    zr = K
    n_pad = -(-(n + 1) // (NS * zr)) * NS * zr
    rps = n_pad // NS
    n_wb = rps // zr

    mesh = plsc.VectorSubcoreMesh(core_axis_name="c", subcore_axis_name="s")

    @functools.partial(
        pl.kernel,
        out_type=jax.ShapeDtypeStruct((NC, n_pad, HALF), jnp.float32),
        mesh=mesh,
        scratch_types=[
            pltpu.VMEM((n_chunk, K), jnp.int32),
            pltpu.VMEM((K,), jnp.int32),
            pltpu.VMEM((K,), jnp.int32),
            pltpu.VMEM((K, HALF), jnp.float32),
            pltpu.VMEM((K, HALF), jnp.float32),
            pltpu.VMEM_SHARED((n_pad, HALF), jnp.float32),
            pltpu.SemaphoreType.DMA,
            pltpu.SemaphoreType.DMA,
            pltpu.SemaphoreType.DMA,
            pltpu.SemaphoreType.DMA,
        ],
    )
    def seg_sum(table_hbm, src_hbm, dst_hbm, out_hbm,
                src_all, dst_a, dst_b, rows_a, rows_b, acc_sh,
                sem_a, sem_b, sem_da, sem_db):
        c = lax.axis_index("c")
        s = lax.axis_index("s")

        def zero_row(i, _):
            for jj in range(HALF // 16):
                rows_a[i, pl.ds(jj * 16, 16)] = jnp.zeros((16,), jnp.float32)
            return 0
        lax.fori_loop(0, zr, zero_row, 0)
        for r in range(n_wb):
            pltpu.async_copy(rows_a, acc_sh.at[pl.ds(s * rps + r * zr, zr)], sem_a)

        pltpu.sync_copy(src_hbm.at[s], src_all)
        pltpu.sync_copy(dst_hbm.at[s, 0], dst_a)
        pltpu.async_copy(dst_hbm.at[s, min(1, n_chunk - 1)], dst_b, sem_db)

        add = c * coeff
        def idx_row(i, _):
            for j in range(K // 16):
                v = src_all[i, pl.ds(j * 16, 16)]
                src_all[i, pl.ds(j * 16, 16)] = v * mult + add
            return 0
        lax.fori_loop(0, n_chunk, idx_row, 0)

        for r in range(n_wb):
            pltpu.make_async_copy(rows_a, acc_sh.at[pl.ds(s * rps + r * zr, zr)],
                                  sem_a).wait()

        def gather_start(k, buf, sem):
            return pltpu.async_copy(table_hbm.at[src_all.at[k]], buf, sem)

        def gather_wait(k, buf, sem):
            pltpu.make_async_copy(table_hbm.at[src_all.at[k]], buf, sem).wait()

        def dst_start(k, buf, sem):
            return pltpu.async_copy(dst_hbm.at[s, k], buf, sem)

        def dst_wait(k, buf, sem):
            pltpu.make_async_copy(dst_hbm.at[s, k], buf, sem).wait()

        def scatter(dbuf, buf):
            pass

        gather_start(0, rows_a, sem_a)
        plsc.subcore_barrier()

        last = n_chunk - 1
        def body(p, _):
            k = 2 * p
            gather_wait(k, rows_a, sem_a)
            gather_start(k + 1, rows_b, sem_b)
            scatter(dst_a, rows_a)
            dst_start(k + 2, dst_a, sem_da)
            gather_wait(k + 1, rows_b, sem_b)
            gather_start(k + 2, rows_a, sem_a)
            dst_wait(k + 1, dst_b, sem_db)
            scatter(dst_b, rows_b)
            dst_start(jnp.minimum(k + 3, last), dst_b, sem_db)
            dst_wait(k + 2, dst_a, sem_da)
            return 0
        lax.fori_loop(0, (n_chunk - 1) // 2, body, 0)
        gather_wait(last, rows_a, sem_a)
        scatter(dst_a, rows_a)
        if n_chunk > 1:
            dst_wait(0, dst_b, sem_db)
        plsc.subcore_barrier()

        bufs = (rows_a, rows_b)
        sems = (sem_a, sem_b)
        pltpu.async_copy(acc_sh.at[pl.ds(s * rps, zr)], rows_a, sem_a)
        for r in range(n_wb):
            row = s * rps + r * zr
            pltpu.make_async_copy(acc_sh.at[pl.ds(row, zr)], bufs[r % 2],
                                  sems[r % 2]).wait()
            if r + 1 < n_wb:
                pltpu.async_copy(acc_sh.at[pl.ds(row + zr, zr)],
                                 bufs[(r + 1) % 2], sems[(r + 1) % 2])
            pltpu.sync_copy(bufs[r % 2], out_hbm.at[c, pl.ds(row, zr)])

    return seg_sum


def _pre_body(x_ref, w_ref, b_ref, out_ref):
    out_ref[...] = (
        jnp.dot(x_ref[...], w_ref[...], preferred_element_type=jnp.float32)
        + b_ref[...]
    )


def _pre_split_body(t_ref, w_ref, b_ref, out_ref):
    w = w_ref[...]
    out_ref[...] = (
        jnp.dot(t_ref[0], w[:HALF, :], preferred_element_type=jnp.float32)
        + jnp.dot(t_ref[1], w[HALF:, :], preferred_element_type=jnp.float32)
        + b_ref[...]
    )


def _combine_relu_body(agg_ref, pre_ref, wrel_ref, out_ref):
    wrel = wrel_ref[...]
    h = jnp.dot(agg_ref[0], wrel[:HALF, :], preferred_element_type=jnp.float32)
    h += jnp.dot(agg_ref[1], wrel[HALF:, :], preferred_element_type=jnp.float32)
    h += pre_ref[...]
    t = jnp.maximum(h, 0.0)
    out_ref[0] = t[:, :HALF]
    out_ref[1] = t[:, HALF:]


def _combine_body(agg_ref, pre_ref, wrel_ref, out_ref):
    wrel = wrel_ref[...]
    h = jnp.dot(agg_ref[0], wrel[:HALF, :], preferred_element_type=jnp.float32)
    h += jnp.dot(agg_ref[1], wrel[HALF:, :], preferred_element_type=jnp.float32)
    h += pre_ref[...]
    out_ref[...] = h


def _tc_pre(x, w, b, bn):
    n, d = x.shape
    return pl.pallas_call(
        _pre_body,
        grid=(n // bn,),
        in_specs=[
            pl.BlockSpec((bn, d), lambda i: (i, 0)),
            pl.BlockSpec((d, d), lambda i: (0, 0)),
            pl.BlockSpec((1, d), lambda i: (0, 0)),
        ],
        out_specs=pl.BlockSpec((bn, d), lambda i: (i, 0)),
        out_shape=jax.ShapeDtypeStruct((n, d), jnp.float32),
    )(x, w, b.reshape(1, d))


def _tc_pre_split(t_split, w, b, bn):
    n = t_split.shape[1]
    d = 2 * HALF
    return pl.pallas_call(
        _pre_split_body,
        grid=(n // bn,),
        in_specs=[
            pl.BlockSpec((2, bn, HALF), lambda i: (0, i, 0)),
            pl.BlockSpec((d, d), lambda i: (0, 0)),
            pl.BlockSpec((1, d), lambda i: (0, 0)),
        ],
        out_specs=pl.BlockSpec((bn, d), lambda i: (i, 0)),
        out_shape=jax.ShapeDtypeStruct((n, d), jnp.float32),
    )(t_split, w, b.reshape(1, d))


def _tc_combine(agg, pre, wrel, bn, relu):
    n, d = pre.shape
    body = _combine_relu_body if relu else _combine_body
    if relu:
        out_shape = jax.ShapeDtypeStruct((2, n, HALF), jnp.float32)
        out_specs = pl.BlockSpec((2, bn, HALF), lambda i: (0, i, 0))
    else:
        out_shape = jax.ShapeDtypeStruct((n, d), jnp.float32)
        out_specs = pl.BlockSpec((bn, d), lambda i: (i, 0))
    return pl.pallas_call(
        body,
        grid=(n // bn,),
        in_specs=[
            pl.BlockSpec((2, bn, HALF), lambda i: (0, i, 0)),
            pl.BlockSpec((bn, d), lambda i: (i, 0)),
            pl.BlockSpec((d, d), lambda i: (0, 0)),
        ],
        out_specs=out_specs,
        out_shape=out_shape,
    )(agg, pre, wrel)


def kernel(x, edge_index, W1_rel, W1_root, b1, W2_rel, W2_root, b2):
    n, d = x.shape
    e = edge_index.shape[1]
    n_chunk = e // (NS * K)
    src = edge_index[0].reshape(NS, n_chunk, K)
    dst = edge_index[1].reshape(NS, n_chunk, K)
    bn = 1000

    x2 = x.reshape(2 * n, HALF)
    pre1 = _tc_pre(x, W1_root, b1, bn)
    agg1 = _make_seg_sum(n, e, 2, 1)(x2, src, dst)[:, :n]
    t_split = _tc_combine(agg1, pre1, W1_rel, bn, relu=True)

    t2 = t_split.reshape(2 * n, HALF)
    pre2 = _tc_pre_split(t_split, W2_root, b2, bn)
    agg2 = _make_seg_sum(n, e, 1, n)(t2, src, dst)[:, :n]
    return _tc_combine(agg2, pre2, W2_rel, bn, relu=False)


# speedup vs baseline: 1.8198x; 1.4058x over previous
import functools

import jax
import jax.numpy as jnp
from jax import lax
from jax.experimental import pallas as pl
from jax.experimental.pallas import tpu as pltpu
from jax.experimental.pallas import tpu_sc as plsc

NS = 16
NC = 2
K = 80
HALF = 128


def _make_seg_sum(n, e, mult, coeff):
    eps = e // NS
    n_chunk = eps // K
    assert n_chunk % 3 == 2 and n_chunk >= 5, "3-deep pipeline layout"
    zr = K
    n_pad = -(-(n + 1) // (NS * zr)) * NS * zr
    rps = n_pad // NS
    n_wb = rps // zr

    mesh = plsc.VectorSubcoreMesh(core_axis_name="c", subcore_axis_name="s")

    @functools.partial(
        pl.kernel,
        out_type=jax.ShapeDtypeStruct((NC, n_pad, HALF), jnp.float32),
        mesh=mesh,
        scratch_types=[
            pltpu.VMEM((n_chunk, K), jnp.int32),
            pltpu.VMEM((K,), jnp.int32),
            pltpu.VMEM((K,), jnp.int32),
            pltpu.VMEM((K,), jnp.int32),
            pltpu.VMEM((K, HALF), jnp.float32),
            pltpu.VMEM((K, HALF), jnp.float32),
            pltpu.VMEM((K, HALF), jnp.float32),
            pltpu.VMEM_SHARED((n_pad, HALF), jnp.float32),
            pltpu.SemaphoreType.DMA,
            pltpu.SemaphoreType.DMA,
            pltpu.SemaphoreType.DMA,
            pltpu.SemaphoreType.DMA,
            pltpu.SemaphoreType.DMA,
            pltpu.SemaphoreType.DMA,
        ],
    )
    def seg_sum(table_hbm, src_hbm, dst_hbm, out_hbm,
                src_all, dst_a, dst_b, dst_c, rows_a, rows_b, rows_c, acc_sh,
                sem_a, sem_b, sem_c, sem_da, sem_db, sem_dc):
        c = lax.axis_index("c")
        s = lax.axis_index("s")

        def zero_row(i, _):
            for jj in range(HALF // 16):
                rows_a[i, pl.ds(jj * 16, 16)] = jnp.zeros((16,), jnp.float32)
            return 0
        lax.fori_loop(0, zr, zero_row, 0)
        for r in range(n_wb):
            pltpu.async_copy(rows_a, acc_sh.at[pl.ds(s * rps + r * zr, zr)], sem_a)

        pltpu.sync_copy(src_hbm.at[s], src_all)
        pltpu.async_copy(dst_hbm.at[s, 0], dst_a, sem_da)
        pltpu.async_copy(dst_hbm.at[s, 1], dst_b, sem_db)

        add = c * coeff
        def idx_row(i, _):
            for j in range(K // 16):
                v = src_all[i, pl.ds(j * 16, 16)]
                src_all[i, pl.ds(j * 16, 16)] = v * mult + add
            return 0
        lax.fori_loop(0, n_chunk, idx_row, 0)

        for r in range(n_wb):
            pltpu.make_async_copy(rows_a, acc_sh.at[pl.ds(s * rps + r * zr, zr)],
                                  sem_a).wait()

        def gather_start(k, buf, sem):
            return pltpu.async_copy(table_hbm.at[src_all.at[k]], buf, sem)

        def gather_wait(k, buf, sem):
            pltpu.make_async_copy(table_hbm.at[src_all.at[k]], buf, sem).wait()

        def dst_start(k, buf, sem):
            return pltpu.async_copy(dst_hbm.at[s, k], buf, sem)

        def dst_wait(k, buf, sem):
            pltpu.make_async_copy(dst_hbm.at[s, k], buf, sem).wait()

        def scatter(dbuf, buf):
            pltpu.sync_copy(buf, acc_sh.at[dbuf], add=True)

        gather_start(0, rows_a, sem_a)
        gather_start(1, rows_b, sem_b)
        plsc.subcore_barrier()

        def body(p, _):
            k = 3 * p
            dst_start(k + 2, dst_c, sem_dc)
            gather_wait(k, rows_a, sem_a)
            gather_start(k + 2, rows_c, sem_c)
            dst_wait(k, dst_a, sem_da)
            scatter(dst_a, rows_a)
            dst_start(k + 3, dst_a, sem_da)
            gather_wait(k + 1, rows_b, sem_b)
            gather_start(k + 3, rows_a, sem_a)
            dst_wait(k + 1, dst_b, sem_db)
            scatter(dst_b, rows_b)
            dst_start(k + 4, dst_b, sem_db)
            gather_wait(k + 2, rows_c, sem_c)
            gather_start(k + 4, rows_b, sem_b)
            dst_wait(k + 2, dst_c, sem_dc)
            scatter(dst_c, rows_c)
            return 0
        lax.fori_loop(0, (n_chunk - 2) // 3, body, 0)
        last = n_chunk - 1
        gather_wait(last - 1, rows_a, sem_a)
        dst_wait(0, dst_a, sem_da)
        scatter(dst_a, rows_a)
        gather_wait(last, rows_b, sem_b)
        dst_wait(0, dst_b, sem_db)
        scatter(dst_b, rows_b)
        plsc.subcore_barrier()

        bufs = (rows_a, rows_b)
        sems = (sem_a, sem_b)
        pltpu.async_copy(acc_sh.at[pl.ds(s * rps, zr)], rows_a, sem_a)
        for r in range(n_wb):
            row = s * rps + r * zr
            pltpu.make_async_copy(acc_sh.at[pl.ds(row, zr)], bufs[r % 2],
                                  sems[r % 2]).wait()
            if r + 1 < n_wb:
                pltpu.async_copy(acc_sh.at[pl.ds(row + zr, zr)],
                                 bufs[(r + 1) % 2], sems[(r + 1) % 2])
            pltpu.sync_copy(bufs[r % 2], out_hbm.at[c, pl.ds(row, zr)])

    return seg_sum


def _pre_body(x_ref, w_ref, b_ref, out_ref):
    out_ref[...] = (
        jnp.dot(x_ref[...], w_ref[...], preferred_element_type=jnp.float32)
        + b_ref[...]
    )


def _pre_split_body(t_ref, w_ref, b_ref, out_ref):
    w = w_ref[...]
    out_ref[...] = (
        jnp.dot(t_ref[0], w[:HALF, :], preferred_element_type=jnp.float32)
        + jnp.dot(t_ref[1], w[HALF:, :], preferred_element_type=jnp.float32)
        + b_ref[...]
    )


def _combine_relu_body(agg_ref, pre_ref, wrel_ref, out_ref):
    wrel = wrel_ref[...]
    h = jnp.dot(agg_ref[0], wrel[:HALF, :], preferred_element_type=jnp.float32)
    h += jnp.dot(agg_ref[1], wrel[HALF:, :], preferred_element_type=jnp.float32)
    h += pre_ref[...]
    t = jnp.maximum(h, 0.0)
    out_ref[0] = t[:, :HALF]
    out_ref[1] = t[:, HALF:]


def _combine_body(agg_ref, pre_ref, wrel_ref, out_ref):
    wrel = wrel_ref[...]
    h = jnp.dot(agg_ref[0], wrel[:HALF, :], preferred_element_type=jnp.float32)
    h += jnp.dot(agg_ref[1], wrel[HALF:, :], preferred_element_type=jnp.float32)
    h += pre_ref[...]
    out_ref[...] = h


def _tc_pre(x, w, b, bn):
    n, d = x.shape
    return pl.pallas_call(
        _pre_body,
        grid=(n // bn,),
        in_specs=[
            pl.BlockSpec((bn, d), lambda i: (i, 0)),
            pl.BlockSpec((d, d), lambda i: (0, 0)),
            pl.BlockSpec((1, d), lambda i: (0, 0)),
        ],
        out_specs=pl.BlockSpec((bn, d), lambda i: (i, 0)),
        out_shape=jax.ShapeDtypeStruct((n, d), jnp.float32),
    )(x, w, b.reshape(1, d))


def _tc_pre_split(t_split, w, b, bn):
    n = t_split.shape[1]
    d = 2 * HALF
    return pl.pallas_call(
        _pre_split_body,
        grid=(n // bn,),
        in_specs=[
            pl.BlockSpec((2, bn, HALF), lambda i: (0, i, 0)),
            pl.BlockSpec((d, d), lambda i: (0, 0)),
            pl.BlockSpec((1, d), lambda i: (0, 0)),
        ],
        out_specs=pl.BlockSpec((bn, d), lambda i: (i, 0)),
        out_shape=jax.ShapeDtypeStruct((n, d), jnp.float32),
    )(t_split, w, b.reshape(1, d))


def _tc_combine(agg, pre, wrel, bn, relu):
    n, d = pre.shape
    body = _combine_relu_body if relu else _combine_body
    if relu:
        out_shape = jax.ShapeDtypeStruct((2, n, HALF), jnp.float32)
        out_specs = pl.BlockSpec((2, bn, HALF), lambda i: (0, i, 0))
    else:
        out_shape = jax.ShapeDtypeStruct((n, d), jnp.float32)
        out_specs = pl.BlockSpec((bn, d), lambda i: (i, 0))
    return pl.pallas_call(
        body,
        grid=(n // bn,),
        in_specs=[
            pl.BlockSpec((2, bn, HALF), lambda i: (0, i, 0)),
            pl.BlockSpec((bn, d), lambda i: (i, 0)),
            pl.BlockSpec((d, d), lambda i: (0, 0)),
        ],
        out_specs=out_specs,
        out_shape=out_shape,
    )(agg, pre, wrel)


def kernel(x, edge_index, W1_rel, W1_root, b1, W2_rel, W2_root, b2):
    n, d = x.shape
    e = edge_index.shape[1]
    n_chunk = e // (NS * K)
    src = edge_index[0].reshape(NS, n_chunk, K)
    dst = edge_index[1].reshape(NS, n_chunk, K)
    bn = 1000

    x2 = x.reshape(2 * n, HALF)
    pre1 = _tc_pre(x, W1_root, b1, bn)
    agg1 = _make_seg_sum(n, e, 2, 1)(x2, src, dst)[:, :n]
    t_split = _tc_combine(agg1, pre1, W1_rel, bn, relu=True)

    t2 = t_split.reshape(2 * n, HALF)
    pre2 = _tc_pre_split(t_split, W2_root, b2, bn)
    agg2 = _make_seg_sum(n, e, 1, n)(t2, src, dst)[:, :n]
    return _tc_combine(agg2, pre2, W2_rel, bn, relu=False)


# speedup vs baseline: 1.8294x; 1.0053x over previous
import functools

import jax
import jax.numpy as jnp
from jax import lax
from jax.experimental import pallas as pl
from jax.experimental.pallas import tpu as pltpu
from jax.experimental.pallas import tpu_sc as plsc

NS = 16
NC = 2
K = 80
HALF = 128


def _make_seg_sum(n, e, mult, coeff):
    eps = e // NS
    n_chunk = eps // K
    assert n_chunk % 3 == 2 and n_chunk >= 5, "3-deep pipeline layout"
    zr = K
    n_pad = -(-(n + 1) // (NS * zr)) * NS * zr
    rps = n_pad // NS
    n_wb = rps // zr

    mesh = plsc.VectorSubcoreMesh(core_axis_name="c", subcore_axis_name="s")

    @functools.partial(
        pl.kernel,
        out_type=jax.ShapeDtypeStruct((NC, n_pad, HALF), jnp.float32),
        mesh=mesh,
        scratch_types=[
            pltpu.VMEM((n_chunk, K), jnp.int32),
            pltpu.VMEM((K,), jnp.int32),
            pltpu.VMEM((K,), jnp.int32),
            pltpu.VMEM((K,), jnp.int32),
            pltpu.VMEM((K, HALF), jnp.float32),
            pltpu.VMEM((K, HALF), jnp.float32),
            pltpu.VMEM((K, HALF), jnp.float32),
            pltpu.VMEM_SHARED((n_pad, HALF), jnp.float32),
            pltpu.SemaphoreType.DMA,
            pltpu.SemaphoreType.DMA,
            pltpu.SemaphoreType.DMA,
            pltpu.SemaphoreType.DMA,
            pltpu.SemaphoreType.DMA,
            pltpu.SemaphoreType.DMA,
            pltpu.SemaphoreType.DMA,
            pltpu.SemaphoreType.DMA,
            pltpu.SemaphoreType.DMA,
        ],
    )
    def seg_sum(table_hbm, src_hbm, dst_hbm, out_hbm,
                src_all, dst_a, dst_b, dst_c, rows_a, rows_b, rows_c, acc_sh,
                sem_a, sem_a2, sem_b, sem_b2, sem_c, sem_c2,
                sem_da, sem_db, sem_dc):
        c = lax.axis_index("c")
        s = lax.axis_index("s")

        def zero_row(i, _):
            for jj in range(HALF // 16):
                rows_a[i, pl.ds(jj * 16, 16)] = jnp.zeros((16,), jnp.float32)
            return 0
        lax.fori_loop(0, zr, zero_row, 0)
        for r in range(n_wb):
            pltpu.async_copy(rows_a, acc_sh.at[pl.ds(s * rps + r * zr, zr)], sem_a)

        pltpu.sync_copy(src_hbm.at[s], src_all)
        pltpu.async_copy(dst_hbm.at[s, 0], dst_a, sem_da)
        pltpu.async_copy(dst_hbm.at[s, 1], dst_b, sem_db)

        add = c * coeff
        def idx_row(i, _):
            for j in range(K // 16):
                v = src_all[i, pl.ds(j * 16, 16)]
                src_all[i, pl.ds(j * 16, 16)] = v * mult + add
            return 0
        lax.fori_loop(0, n_chunk, idx_row, 0)

        for r in range(n_wb):
            pltpu.make_async_copy(rows_a, acc_sh.at[pl.ds(s * rps + r * zr, zr)],
                                  sem_a).wait()

        h2 = K // 2

        def gather_start(k, buf, sems):
            pltpu.async_copy(table_hbm.at[src_all.at[k, pl.ds(0, h2)]],
                             buf.at[pl.ds(0, h2)], sems[0])
            pltpu.async_copy(table_hbm.at[src_all.at[k, pl.ds(h2, h2)]],
                             buf.at[pl.ds(h2, h2)], sems[1])

        def gather_wait(k, buf, sems):
            pltpu.make_async_copy(table_hbm.at[src_all.at[k, pl.ds(0, h2)]],
                                  buf.at[pl.ds(0, h2)], sems[0]).wait()
            pltpu.make_async_copy(table_hbm.at[src_all.at[k, pl.ds(h2, h2)]],
                                  buf.at[pl.ds(h2, h2)], sems[1]).wait()

        def dst_start(k, buf, sem):
            return pltpu.async_copy(dst_hbm.at[s, k], buf, sem)

        def dst_wait(k, buf, sem):
            pltpu.make_async_copy(dst_hbm.at[s, k], buf, sem).wait()

        def scatter(dbuf, buf):
            pltpu.sync_copy(buf, acc_sh.at[dbuf], add=True)

        gather_start(0, rows_a, (sem_a, sem_a2))
        gather_start(1, rows_b, (sem_b, sem_b2))
        plsc.subcore_barrier()

        def body(p, _):
            k = 3 * p
            dst_start(k + 2, dst_c, sem_dc)
            gather_wait(k, rows_a, (sem_a, sem_a2))
            gather_start(k + 2, rows_c, (sem_c, sem_c2))
            dst_wait(k, dst_a, sem_da)
            scatter(dst_a, rows_a)
            dst_start(k + 3, dst_a, sem_da)
            gather_wait(k + 1, rows_b, (sem_b, sem_b2))
            gather_start(k + 3, rows_a, (sem_a, sem_a2))
            dst_wait(k + 1, dst_b, sem_db)
            scatter(dst_b, rows_b)
            dst_start(k + 4, dst_b, sem_db)
            gather_wait(k + 2, rows_c, (sem_c, sem_c2))
            gather_start(k + 4, rows_b, (sem_b, sem_b2))
            dst_wait(k + 2, dst_c, sem_dc)
            scatter(dst_c, rows_c)
            return 0
        lax.fori_loop(0, (n_chunk - 2) // 3, body, 0)
        last = n_chunk - 1
        gather_wait(last - 1, rows_a, (sem_a, sem_a2))
        dst_wait(0, dst_a, sem_da)
        scatter(dst_a, rows_a)
        gather_wait(last, rows_b, (sem_b, sem_b2))
        dst_wait(0, dst_b, sem_db)
        scatter(dst_b, rows_b)
        plsc.subcore_barrier()

        bufs = (rows_a, rows_b)
        sems = (sem_a, sem_b)
        pltpu.async_copy(acc_sh.at[pl.ds(s * rps, zr)], rows_a, sem_a)
        for r in range(n_wb):
            row = s * rps + r * zr
            pltpu.make_async_copy(acc_sh.at[pl.ds(row, zr)], bufs[r % 2],
                                  sems[r % 2]).wait()
            if r + 1 < n_wb:
                pltpu.async_copy(acc_sh.at[pl.ds(row + zr, zr)],
                                 bufs[(r + 1) % 2], sems[(r + 1) % 2])
            pltpu.sync_copy(bufs[r % 2], out_hbm.at[c, pl.ds(row, zr)])

    return seg_sum


def _pre_body(x_ref, w_ref, b_ref, out_ref):
    out_ref[...] = (
        jnp.dot(x_ref[...], w_ref[...], preferred_element_type=jnp.float32)
        + b_ref[...]
    )


def _pre_split_body(t_ref, w_ref, b_ref, out_ref):
    w = w_ref[...]
    out_ref[...] = (
        jnp.dot(t_ref[0], w[:HALF, :], preferred_element_type=jnp.float32)
        + jnp.dot(t_ref[1], w[HALF:, :], preferred_element_type=jnp.float32)
        + b_ref[...]
    )


def _combine_relu_body(agg_ref, pre_ref, wrel_ref, out_ref):
    wrel = wrel_ref[...]
    h = jnp.dot(agg_ref[0], wrel[:HALF, :], preferred_element_type=jnp.float32)
    h += jnp.dot(agg_ref[1], wrel[HALF:, :], preferred_element_type=jnp.float32)
    h += pre_ref[...]
    t = jnp.maximum(h, 0.0)
    out_ref[0] = t[:, :HALF]
    out_ref[1] = t[:, HALF:]


def _combine_body(agg_ref, pre_ref, wrel_ref, out_ref):
    wrel = wrel_ref[...]
    h = jnp.dot(agg_ref[0], wrel[:HALF, :], preferred_element_type=jnp.float32)
    h += jnp.dot(agg_ref[1], wrel[HALF:, :], preferred_element_type=jnp.float32)
    h += pre_ref[...]
    out_ref[...] = h


def _tc_pre(x, w, b, bn):
    n, d = x.shape
    return pl.pallas_call(
        _pre_body,
        grid=(n // bn,),
        in_specs=[
            pl.BlockSpec((bn, d), lambda i: (i, 0)),
            pl.BlockSpec((d, d), lambda i: (0, 0)),
            pl.BlockSpec((1, d), lambda i: (0, 0)),
        ],
        out_specs=pl.BlockSpec((bn, d), lambda i: (i, 0)),
        out_shape=jax.ShapeDtypeStruct((n, d), jnp.float32),
    )(x, w, b.reshape(1, d))


def _tc_pre_split(t_split, w, b, bn):
    n = t_split.shape[1]
    d = 2 * HALF
    return pl.pallas_call(
        _pre_split_body,
        grid=(n // bn,),
        in_specs=[
            pl.BlockSpec((2, bn, HALF), lambda i: (0, i, 0)),
            pl.BlockSpec((d, d), lambda i: (0, 0)),
            pl.BlockSpec((1, d), lambda i: (0, 0)),
        ],
        out_specs=pl.BlockSpec((bn, d), lambda i: (i, 0)),
        out_shape=jax.ShapeDtypeStruct((n, d), jnp.float32),
    )(t_split, w, b.reshape(1, d))


def _tc_combine(agg, pre, wrel, bn, relu):
    n, d = pre.shape
    body = _combine_relu_body if relu else _combine_body
    if relu:
        out_shape = jax.ShapeDtypeStruct((2, n, HALF), jnp.float32)
        out_specs = pl.BlockSpec((2, bn, HALF), lambda i: (0, i, 0))
    else:
        out_shape = jax.ShapeDtypeStruct((n, d), jnp.float32)
        out_specs = pl.BlockSpec((bn, d), lambda i: (i, 0))
    return pl.pallas_call(
        body,
        grid=(n // bn,),
        in_specs=[
            pl.BlockSpec((2, bn, HALF), lambda i: (0, i, 0)),
            pl.BlockSpec((bn, d), lambda i: (i, 0)),
            pl.BlockSpec((d, d), lambda i: (0, 0)),
        ],
        out_specs=out_specs,
        out_shape=out_shape,
    )(agg, pre, wrel)


def kernel(x, edge_index, W1_rel, W1_root, b1, W2_rel, W2_root, b2):
    n, d = x.shape
    e = edge_index.shape[1]
    n_chunk = e // (NS * K)
    src = edge_index[0].reshape(NS, n_chunk, K)
    dst = edge_index[1].reshape(NS, n_chunk, K)
    bn = 1000

    x2 = x.reshape(2 * n, HALF)
    pre1 = _tc_pre(x, W1_root, b1, bn)
    agg1 = _make_seg_sum(n, e, 2, 1)(x2, src, dst)[:, :n]
    t_split = _tc_combine(agg1, pre1, W1_rel, bn, relu=True)

    t2 = t_split.reshape(2 * n, HALF)
    pre2 = _tc_pre_split(t_split, W2_root, b2, bn)
    agg2 = _make_seg_sum(n, e, 1, n)(t2, src, dst)[:, :n]
    return _tc_combine(agg2, pre2, W2_rel, bn, relu=False)


# speedup vs baseline: 1.8453x; 1.0087x over previous
import functools

import jax
import jax.numpy as jnp
from jax import lax
from jax.experimental import pallas as pl
from jax.experimental.pallas import tpu as pltpu
from jax.experimental.pallas import tpu_sc as plsc

NS = 16
NC = 2
K = 80
HALF = 128


def _make_seg_sum(n, e, mult, coeff):
    eps = e // NS
    n_chunk = eps // K
    assert n_chunk % 3 == 2 and n_chunk >= 5, "3-deep pipeline layout"
    zr = K
    n_pad = -(-(n + 1) // (NS * zr)) * NS * zr
    rps = n_pad // NS
    n_wb = rps // zr

    mesh = plsc.VectorSubcoreMesh(core_axis_name="c", subcore_axis_name="s")

    @functools.partial(
        pl.kernel,
        out_type=jax.ShapeDtypeStruct((NC, n_pad, HALF), jnp.float32),
        mesh=mesh,
        scratch_types=[
            pltpu.VMEM((n_chunk, K), jnp.int32),
            pltpu.VMEM((K,), jnp.int32),
            pltpu.VMEM((K,), jnp.int32),
            pltpu.VMEM((K,), jnp.int32),
            pltpu.VMEM((K, HALF), jnp.float32),
            pltpu.VMEM((K, HALF), jnp.float32),
            pltpu.VMEM((K, HALF), jnp.float32),
            pltpu.VMEM_SHARED((n_pad, HALF), jnp.float32),
            pltpu.SemaphoreType.DMA,
            pltpu.SemaphoreType.DMA,
            pltpu.SemaphoreType.DMA,
            pltpu.SemaphoreType.DMA,
            pltpu.SemaphoreType.DMA,
            pltpu.SemaphoreType.DMA,
            pltpu.SemaphoreType.DMA,
            pltpu.SemaphoreType.DMA,
            pltpu.SemaphoreType.DMA,
        ],
    )
    def seg_sum(table_hbm, src_hbm, dst_hbm, out_hbm,
                src_all, dst_a, dst_b, dst_c, rows_a, rows_b, rows_c, acc_sh,
                sem_a, sem_a2, sem_b, sem_b2, sem_c, sem_c2,
                sem_da, sem_db, sem_dc):
        c = lax.axis_index("c")
        s = lax.axis_index("s")

        def zero_row(i, _):
            for jj in range(HALF // 16):
                rows_a[i, pl.ds(jj * 16, 16)] = jnp.zeros((16,), jnp.float32)
            return 0
        lax.fori_loop(0, zr, zero_row, 0)
        for r in range(n_wb):
            pltpu.async_copy(rows_a, acc_sh.at[pl.ds(s * rps + r * zr, zr)], sem_a)

        pltpu.sync_copy(src_hbm.at[s], src_all)
        pltpu.async_copy(dst_hbm.at[s, 0], dst_a, sem_da)
        pltpu.async_copy(dst_hbm.at[s, 1], dst_b, sem_db)

        add = c * coeff
        def idx_row(i, _):
            for j in range(K // 16):
                v = src_all[i, pl.ds(j * 16, 16)]
                src_all[i, pl.ds(j * 16, 16)] = v * mult + add
            return 0
        lax.fori_loop(0, n_chunk, idx_row, 0)

        for r in range(n_wb):
            pltpu.make_async_copy(rows_a, acc_sh.at[pl.ds(s * rps + r * zr, zr)],
                                  sem_a).wait()

        h2 = K // 2

        def gather_start(k, buf, sems):
            pltpu.async_copy(table_hbm.at[src_all.at[k, pl.ds(0, h2)]],
                             buf.at[pl.ds(0, h2)], sems[0])
            pltpu.async_copy(table_hbm.at[src_all.at[k, pl.ds(h2, h2)]],
                             buf.at[pl.ds(h2, h2)], sems[1])

        def gather_wait(k, buf, sems):
            pltpu.make_async_copy(table_hbm.at[src_all.at[k, pl.ds(0, h2)]],
                                  buf.at[pl.ds(0, h2)], sems[0]).wait()
            pltpu.make_async_copy(table_hbm.at[src_all.at[k, pl.ds(h2, h2)]],
                                  buf.at[pl.ds(h2, h2)], sems[1]).wait()

        def dst_start(k, buf, sem):
            return pltpu.async_copy(dst_hbm.at[s, k], buf, sem)

        def dst_wait(k, buf, sem):
            pltpu.make_async_copy(dst_hbm.at[s, k], buf, sem).wait()

        def scatter(dbuf, buf):
            pltpu.sync_copy(buf, acc_sh.at[dbuf], add=True)

        gather_start(0, rows_a, (sem_a, sem_a2))
        gather_start(1, rows_b, (sem_b, sem_b2))
        plsc.subcore_barrier()

        def body(p, _):
            k = 3 * p
            dst_start(k + 2, dst_c, sem_dc)
            gather_wait(k, rows_a, (sem_a, sem_a2))
            gather_start(k + 2, rows_c, (sem_c, sem_c2))
            dst_wait(k, dst_a, sem_da)
            scatter(dst_a, rows_a)
            dst_start(k + 3, dst_a, sem_da)
            gather_wait(k + 1, rows_b, (sem_b, sem_b2))
            gather_start(k + 3, rows_a, (sem_a, sem_a2))
            dst_wait(k + 1, dst_b, sem_db)
            scatter(dst_b, rows_b)
            dst_start(k + 4, dst_b, sem_db)
            gather_wait(k + 2, rows_c, (sem_c, sem_c2))
            gather_start(k + 4, rows_b, (sem_b, sem_b2))
            dst_wait(k + 2, dst_c, sem_dc)
            scatter(dst_c, rows_c)
            return 0
        lax.fori_loop(0, (n_chunk - 2) // 3, body, 0)
        last = n_chunk - 1
        gather_wait(last - 1, rows_a, (sem_a, sem_a2))
        dst_wait(0, dst_a, sem_da)
        scatter(dst_a, rows_a)
        gather_wait(last, rows_b, (sem_b, sem_b2))
        dst_wait(0, dst_b, sem_db)
        scatter(dst_b, rows_b)
        plsc.subcore_barrier()

        bufs = (rows_a, rows_b)
        sems = (sem_a, sem_b)
        pltpu.async_copy(acc_sh.at[pl.ds(s * rps, zr)], rows_a, sem_a)
        for r in range(n_wb):
            row = s * rps + r * zr
            pltpu.make_async_copy(acc_sh.at[pl.ds(row, zr)], bufs[r % 2],
                                  sems[r % 2]).wait()
            if r + 1 < n_wb:
                pltpu.async_copy(acc_sh.at[pl.ds(row + zr, zr)],
                                 bufs[(r + 1) % 2], sems[(r + 1) % 2])
            pltpu.sync_copy(bufs[r % 2], out_hbm.at[c, pl.ds(row, zr)])

    return seg_sum


def _pre_body(x_ref, w_ref, b_ref, out_ref):
    out_ref[...] = (
        jnp.dot(x_ref[...], w_ref[...], preferred_element_type=jnp.float32)
        + b_ref[...]
    )


def _pre_split_body(t_ref, w_ref, b_ref, out_ref):
    w = w_ref[...]
    out_ref[...] = (
        jnp.dot(t_ref[0], w[:HALF, :], preferred_element_type=jnp.float32)
        + jnp.dot(t_ref[1], w[HALF:, :], preferred_element_type=jnp.float32)
        + b_ref[...]
    )


def _combine_relu_body(agg_ref, pre_ref, wrel_ref, out_ref):
    wrel = wrel_ref[...]
    h = jnp.dot(agg_ref[0], wrel[:HALF, :], preferred_element_type=jnp.float32)
    h += jnp.dot(agg_ref[1], wrel[HALF:, :], preferred_element_type=jnp.float32)
    h += pre_ref[...]
    t = jnp.maximum(h, 0.0)
    out_ref[0] = t[:, :HALF]
    out_ref[1] = t[:, HALF:]


def _combine_body(agg_ref, pre_ref, wrel_ref, out_ref):
    wrel = wrel_ref[...]
    h = jnp.dot(agg_ref[0], wrel[:HALF, :], preferred_element_type=jnp.float32)
    h += jnp.dot(agg_ref[1], wrel[HALF:, :], preferred_element_type=jnp.float32)
    h += pre_ref[...]
    out_ref[...] = h


def _tc_pre(x, w, b, bn):
    n, d = x.shape
    return pl.pallas_call(
        _pre_body,
        grid=(n // bn,),
        in_specs=[
            pl.BlockSpec((bn, d), lambda i: (i, 0)),
            pl.BlockSpec((d, d), lambda i: (0, 0)),
            pl.BlockSpec((1, d), lambda i: (0, 0)),
        ],
        out_specs=pl.BlockSpec((bn, d), lambda i: (i, 0)),
        out_shape=jax.ShapeDtypeStruct((n, d), jnp.float32),
    )(x, w, b.reshape(1, d))


def _tc_pre_split(t_split, w, b, bn):
    n = t_split.shape[1]
    d = 2 * HALF
    return pl.pallas_call(
        _pre_split_body,
        grid=(n // bn,),
        in_specs=[
            pl.BlockSpec((2, bn, HALF), lambda i: (0, i, 0)),
            pl.BlockSpec((d, d), lambda i: (0, 0)),
            pl.BlockSpec((1, d), lambda i: (0, 0)),
        ],
        out_specs=pl.BlockSpec((bn, d), lambda i: (i, 0)),
        out_shape=jax.ShapeDtypeStruct((n, d), jnp.float32),
    )(t_split, w, b.reshape(1, d))


def _tc_combine(agg, pre, wrel, bn, relu):
    n, d = pre.shape
    body = _combine_relu_body if relu else _combine_body
    if relu:
        out_shape = jax.ShapeDtypeStruct((2, n, HALF), jnp.float32)
        out_specs = pl.BlockSpec((2, bn, HALF), lambda i: (0, i, 0))
    else:
        out_shape = jax.ShapeDtypeStruct((n, d), jnp.float32)
        out_specs = pl.BlockSpec((bn, d), lambda i: (i, 0))
    return pl.pallas_call(
        body,
        grid=(n // bn,),
        in_specs=[
            pl.BlockSpec((2, bn, HALF), lambda i: (0, i, 0)),
            pl.BlockSpec((bn, d), lambda i: (i, 0)),
            pl.BlockSpec((d, d), lambda i: (0, 0)),
        ],
        out_specs=out_specs,
        out_shape=out_shape,
    )(agg, pre, wrel)


def _layer1_body(agg_ref, x_ref, wrel_ref, wroot_ref, b_ref, out_ref):
    wrel = wrel_ref[...]
    h = jnp.dot(agg_ref[0], wrel[:HALF, :], preferred_element_type=jnp.float32)
    h += jnp.dot(agg_ref[1], wrel[HALF:, :], preferred_element_type=jnp.float32)
    h += jnp.dot(x_ref[...], wroot_ref[...], preferred_element_type=jnp.float32)
    h += b_ref[...]
    t = jnp.maximum(h, 0.0)
    out_ref[0] = t[:, :HALF]
    out_ref[1] = t[:, HALF:]


def _layer2_body(agg_ref, t_ref, wrel_ref, wroot_ref, b_ref, out_ref):
    wrel = wrel_ref[...]
    wroot = wroot_ref[...]
    h = jnp.dot(agg_ref[0], wrel[:HALF, :], preferred_element_type=jnp.float32)
    h += jnp.dot(agg_ref[1], wrel[HALF:, :], preferred_element_type=jnp.float32)
    h += jnp.dot(t_ref[0], wroot[:HALF, :], preferred_element_type=jnp.float32)
    h += jnp.dot(t_ref[1], wroot[HALF:, :], preferred_element_type=jnp.float32)
    h += b_ref[...]
    out_ref[...] = h


def _tc_layer1(agg, x, wrel, wroot, b, bn):
    n, d = x.shape
    return pl.pallas_call(
        _layer1_body,
        grid=(n // bn,),
        in_specs=[
            pl.BlockSpec((2, bn, HALF), lambda i: (0, i, 0)),
            pl.BlockSpec((bn, d), lambda i: (i, 0)),
            pl.BlockSpec((d, d), lambda i: (0, 0)),
            pl.BlockSpec((d, d), lambda i: (0, 0)),
            pl.BlockSpec((1, d), lambda i: (0, 0)),
        ],
        out_specs=pl.BlockSpec((2, bn, HALF), lambda i: (0, i, 0)),
        out_shape=jax.ShapeDtypeStruct((2, n, HALF), jnp.float32),
    )(agg, x, wrel, wroot, b.reshape(1, d))


def _tc_layer2(agg, t_split, wrel, wroot, b, bn):
    n = agg.shape[1]
    d = 2 * HALF
    return pl.pallas_call(
        _layer2_body,
        grid=(n // bn,),
        in_specs=[
            pl.BlockSpec((2, bn, HALF), lambda i: (0, i, 0)),
            pl.BlockSpec((2, bn, HALF), lambda i: (0, i, 0)),
            pl.BlockSpec((d, d), lambda i: (0, 0)),
            pl.BlockSpec((d, d), lambda i: (0, 0)),
            pl.BlockSpec((1, d), lambda i: (0, 0)),
        ],
        out_specs=pl.BlockSpec((bn, d), lambda i: (i, 0)),
        out_shape=jax.ShapeDtypeStruct((n, d), jnp.float32),
    )(agg, t_split, wrel, wroot, b.reshape(1, d))


def kernel(x, edge_index, W1_rel, W1_root, b1, W2_rel, W2_root, b2):
    n, d = x.shape
    e = edge_index.shape[1]
    n_chunk = e // (NS * K)
    src = edge_index[0].reshape(NS, n_chunk, K)
    dst = edge_index[1].reshape(NS, n_chunk, K)
    bn = 1000

    x2 = x.reshape(2 * n, HALF)
    agg1 = _make_seg_sum(n, e, 2, 1)(x2, src, dst)[:, :n]
    t_split = _tc_layer1(agg1, x, W1_rel, W1_root, b1, bn)

    t2 = t_split.reshape(2 * n, HALF)
    agg2 = _make_seg_sum(n, e, 1, n)(t2, src, dst)[:, :n]
    return _tc_layer2(agg2, t_split, W2_rel, W2_root, b2, bn)


# speedup vs baseline: 1.9475x; 1.0554x over previous
import functools

import jax
import jax.numpy as jnp
from jax import lax
from jax.experimental import pallas as pl
from jax.experimental.pallas import tpu as pltpu
from jax.experimental.pallas import tpu_sc as plsc

NS = 16
NC = 2
K = 80
HALF = 128


def _make_seg_sum(n, e, mult, coeff):
    eps = e // NS
    n_chunk = eps // K
    assert n_chunk % 3 == 2 and n_chunk >= 5, "3-deep pipeline layout"
    zr = K
    n_pad = -(-(n + 1) // (NS * zr)) * NS * zr
    rps = n_pad // NS
    n_wb = rps // zr

    mesh = plsc.VectorSubcoreMesh(core_axis_name="c", subcore_axis_name="s")

    @functools.partial(
        pl.kernel,
        out_type=jax.ShapeDtypeStruct((NC, n_pad, HALF), jnp.float32),
        mesh=mesh,
        scratch_types=[
            pltpu.VMEM((n_chunk, K), jnp.int32),
            pltpu.VMEM((K,), jnp.int32),
            pltpu.VMEM((K,), jnp.int32),
            pltpu.VMEM((K,), jnp.int32),
            pltpu.VMEM((K, HALF), jnp.float32),
            pltpu.VMEM((K, HALF), jnp.float32),
            pltpu.VMEM((K, HALF), jnp.float32),
            pltpu.VMEM_SHARED((n_pad, HALF), jnp.float32),
            pltpu.SemaphoreType.DMA,
            pltpu.SemaphoreType.DMA,
            pltpu.SemaphoreType.DMA,
            pltpu.SemaphoreType.DMA,
            pltpu.SemaphoreType.DMA,
            pltpu.SemaphoreType.DMA,
            pltpu.SemaphoreType.DMA,
            pltpu.SemaphoreType.DMA,
            pltpu.SemaphoreType.DMA,
        ],
    )
    def seg_sum(table_hbm, src_hbm, dst_hbm, out_hbm,
                src_all, dst_a, dst_b, dst_c, rows_a, rows_b, rows_c, acc_sh,
                sem_a, sem_a2, sem_b, sem_b2, sem_c, sem_c2,
                sem_da, sem_db, sem_dc):
        c = lax.axis_index("c")
        s = lax.axis_index("s")

        def zero_row(i, _):
            for jj in range(HALF // 16):
                rows_a[i, pl.ds(jj * 16, 16)] = jnp.zeros((16,), jnp.float32)
            return 0
        lax.fori_loop(0, zr, zero_row, 0)
        for r in range(n_wb):
            pltpu.async_copy(rows_a, acc_sh.at[pl.ds(s * rps + r * zr, zr)], sem_a)

        pltpu.sync_copy(src_hbm.at[s], src_all)
        pltpu.async_copy(dst_hbm.at[s, 0], dst_a, sem_da)
        pltpu.async_copy(dst_hbm.at[s, 1], dst_b, sem_db)

        add = c * coeff
        def idx_row(i, _):
            for j in range(K // 16):
                v = src_all[i, pl.ds(j * 16, 16)]
                src_all[i, pl.ds(j * 16, 16)] = v * mult + add
            return 0
        lax.fori_loop(0, n_chunk, idx_row, 0)

        for r in range(n_wb):
            pltpu.make_async_copy(rows_a, acc_sh.at[pl.ds(s * rps + r * zr, zr)],
                                  sem_a).wait()

        h2 = K // 2

        def gather_start(k, buf, sems):
            pltpu.async_copy(table_hbm.at[src_all.at[k, pl.ds(0, h2)]],
                             buf.at[pl.ds(0, h2)], sems[0])
            pltpu.async_copy(table_hbm.at[src_all.at[k, pl.ds(h2, h2)]],
                             buf.at[pl.ds(h2, h2)], sems[1])

        def gather_wait(k, buf, sems):
            pltpu.make_async_copy(table_hbm.at[src_all.at[k, pl.ds(0, h2)]],
                                  buf.at[pl.ds(0, h2)], sems[0]).wait()
            pltpu.make_async_copy(table_hbm.at[src_all.at[k, pl.ds(h2, h2)]],
                                  buf.at[pl.ds(h2, h2)], sems[1]).wait()

        def dst_start(k, buf, sem):
            return pltpu.async_copy(dst_hbm.at[s, k], buf, sem)

        def dst_wait(k, buf, sem):
            pltpu.make_async_copy(dst_hbm.at[s, k], buf, sem).wait()

        def scatter(dbuf, buf):
            pltpu.sync_copy(buf, acc_sh.at[dbuf], add=True)

        gather_start(0, rows_a, (sem_a, sem_a2))
        gather_start(1, rows_b, (sem_b, sem_b2))
        plsc.subcore_barrier()

        def body(p, _):
            k = 3 * p
            dst_start(k + 2, dst_c, sem_dc)
            gather_wait(k, rows_a, (sem_a, sem_a2))
            gather_start(k + 2, rows_c, (sem_c, sem_c2))
            dst_wait(k, dst_a, sem_da)
            scatter(dst_a, rows_a)
            dst_start(k + 3, dst_a, sem_da)
            gather_wait(k + 1, rows_b, (sem_b, sem_b2))
            gather_start(k + 3, rows_a, (sem_a, sem_a2))
            dst_wait(k + 1, dst_b, sem_db)
            scatter(dst_b, rows_b)
            dst_start(k + 4, dst_b, sem_db)
            gather_wait(k + 2, rows_c, (sem_c, sem_c2))
            gather_start(k + 4, rows_b, (sem_b, sem_b2))
            dst_wait(k + 2, dst_c, sem_dc)
            scatter(dst_c, rows_c)
            return 0
        lax.fori_loop(0, (n_chunk - 2) // 3, body, 0)
        last = n_chunk - 1
        gather_wait(last - 1, rows_a, (sem_a, sem_a2))
        dst_wait(0, dst_a, sem_da)
        scatter(dst_a, rows_a)
        gather_wait(last, rows_b, (sem_b, sem_b2))
        dst_wait(0, dst_b, sem_db)
        scatter(dst_b, rows_b)
        plsc.subcore_barrier()

        bufs = (rows_a, rows_b)
        sems = (sem_a, sem_b)
        pltpu.async_copy(acc_sh.at[pl.ds(s * rps, zr)], rows_a, sem_a)
        for r in range(n_wb):
            row = s * rps + r * zr
            pltpu.make_async_copy(acc_sh.at[pl.ds(row, zr)], bufs[r % 2],
                                  sems[r % 2]).wait()
            if r + 1 < n_wb:
                pltpu.async_copy(acc_sh.at[pl.ds(row + zr, zr)],
                                 bufs[(r + 1) % 2], sems[(r + 1) % 2])
            pltpu.sync_copy(bufs[r % 2], out_hbm.at[c, pl.ds(row, zr)])

    return seg_sum


def _pre_body(x_ref, w_ref, b_ref, out_ref):
    out_ref[...] = (
        jnp.dot(x_ref[...], w_ref[...], preferred_element_type=jnp.float32)
        + b_ref[...]
    )


def _pre_split_body(t_ref, w_ref, b_ref, out_ref):
    w = w_ref[...]
    out_ref[...] = (
        jnp.dot(t_ref[0], w[:HALF, :], preferred_element_type=jnp.float32)
        + jnp.dot(t_ref[1], w[HALF:, :], preferred_element_type=jnp.float32)
        + b_ref[...]
    )


def _combine_relu_body(agg_ref, pre_ref, wrel_ref, out_ref):
    wrel = wrel_ref[...]
    h = jnp.dot(agg_ref[0], wrel[:HALF, :], preferred_element_type=jnp.float32)
    h += jnp.dot(agg_ref[1], wrel[HALF:, :], preferred_element_type=jnp.float32)
    h += pre_ref[...]
    t = jnp.maximum(h, 0.0)
    out_ref[0] = t[:, :HALF]
    out_ref[1] = t[:, HALF:]


def _combine_body(agg_ref, pre_ref, wrel_ref, out_ref):
    wrel = wrel_ref[...]
    h = jnp.dot(agg_ref[0], wrel[:HALF, :], preferred_element_type=jnp.float32)
    h += jnp.dot(agg_ref[1], wrel[HALF:, :], preferred_element_type=jnp.float32)
    h += pre_ref[...]
    out_ref[...] = h


def _tc_pre(x, w, b, bn):
    n, d = x.shape
    return pl.pallas_call(
        _pre_body,
        grid=(n // bn,),
        in_specs=[
            pl.BlockSpec((bn, d), lambda i: (i, 0)),
            pl.BlockSpec((d, d), lambda i: (0, 0)),
            pl.BlockSpec((1, d), lambda i: (0, 0)),
        ],
        out_specs=pl.BlockSpec((bn, d), lambda i: (i, 0)),
        out_shape=jax.ShapeDtypeStruct((n, d), jnp.float32),
    )(x, w, b.reshape(1, d))


def _tc_pre_split(t_split, w, b, bn):
    n = t_split.shape[1]
    d = 2 * HALF
    return pl.pallas_call(
        _pre_split_body,
        grid=(n // bn,),
        in_specs=[
            pl.BlockSpec((2, bn, HALF), lambda i: (0, i, 0)),
            pl.BlockSpec((d, d), lambda i: (0, 0)),
            pl.BlockSpec((1, d), lambda i: (0, 0)),
        ],
        out_specs=pl.BlockSpec((bn, d), lambda i: (i, 0)),
        out_shape=jax.ShapeDtypeStruct((n, d), jnp.float32),
    )(t_split, w, b.reshape(1, d))


def _tc_combine(agg, pre, wrel, bn, relu):
    n, d = pre.shape
    body = _combine_relu_body if relu else _combine_body
    if relu:
        out_shape = jax.ShapeDtypeStruct((2, n, HALF), jnp.float32)
        out_specs = pl.BlockSpec((2, bn, HALF), lambda i: (0, i, 0))
    else:
        out_shape = jax.ShapeDtypeStruct((n, d), jnp.float32)
        out_specs = pl.BlockSpec((bn, d), lambda i: (i, 0))
    return pl.pallas_call(
        body,
        grid=(n // bn,),
        in_specs=[
            pl.BlockSpec((2, bn, HALF), lambda i: (0, i, 0)),
            pl.BlockSpec((bn, d), lambda i: (i, 0)),
            pl.BlockSpec((d, d), lambda i: (0, 0)),
        ],
        out_specs=out_specs,
        out_shape=out_shape,
    )(agg, pre, wrel)


def _layer1_body(agg_ref, x_ref, wrel_ref, wroot_ref, b_ref, out_ref):
    wrel = wrel_ref[...]
    h = jnp.dot(agg_ref[0], wrel[:HALF, :], preferred_element_type=jnp.float32)
    h += jnp.dot(agg_ref[1], wrel[HALF:, :], preferred_element_type=jnp.float32)
    h += jnp.dot(x_ref[...], wroot_ref[...], preferred_element_type=jnp.float32)
    h += b_ref[...]
    t = jnp.maximum(h, 0.0)
    out_ref[0] = t[:, :HALF]
    out_ref[1] = t[:, HALF:]


def _layer2_body(agg_ref, t_ref, wrel_ref, wroot_ref, b_ref, out_ref):
    wrel = wrel_ref[...]
    wroot = wroot_ref[...]
    h = jnp.dot(agg_ref[0], wrel[:HALF, :], preferred_element_type=jnp.float32)
    h += jnp.dot(agg_ref[1], wrel[HALF:, :], preferred_element_type=jnp.float32)
    h += jnp.dot(t_ref[0], wroot[:HALF, :], preferred_element_type=jnp.float32)
    h += jnp.dot(t_ref[1], wroot[HALF:, :], preferred_element_type=jnp.float32)
    h += b_ref[...]
    out_ref[...] = h


def _tc_layer1(agg, x, wrel, wroot, b, bn):
    n, d = x.shape
    return pl.pallas_call(
        _layer1_body,
        grid=(n // bn,),
        in_specs=[
            pl.BlockSpec((2, bn, HALF), lambda i: (0, i, 0)),
            pl.BlockSpec((bn, d), lambda i: (i, 0)),
            pl.BlockSpec((d, d), lambda i: (0, 0)),
            pl.BlockSpec((d, d), lambda i: (0, 0)),
            pl.BlockSpec((1, d), lambda i: (0, 0)),
        ],
        out_specs=pl.BlockSpec((2, bn, HALF), lambda i: (0, i, 0)),
        out_shape=jax.ShapeDtypeStruct((2, n, HALF), jnp.float32),
    )(agg, x, wrel, wroot, b.reshape(1, d))


def _tc_layer2(agg, t_split, wrel, wroot, b, bn):
    n = t_split.shape[1]
    d = 2 * HALF
    return pl.pallas_call(
        _layer2_body,
        grid=(n // bn,),
        in_specs=[
            pl.BlockSpec((2, bn, HALF), lambda i: (0, i, 0)),
            pl.BlockSpec((2, bn, HALF), lambda i: (0, i, 0)),
            pl.BlockSpec((d, d), lambda i: (0, 0)),
            pl.BlockSpec((d, d), lambda i: (0, 0)),
            pl.BlockSpec((1, d), lambda i: (0, 0)),
        ],
        out_specs=pl.BlockSpec((bn, d), lambda i: (i, 0)),
        out_shape=jax.ShapeDtypeStruct((n, d), jnp.float32),
    )(agg, t_split, wrel, wroot, b.reshape(1, d))


def kernel(x, edge_index, W1_rel, W1_root, b1, W2_rel, W2_root, b2):
    n, d = x.shape
    e = edge_index.shape[1]
    n_chunk = e // (NS * K)
    src = edge_index[0].reshape(NS, n_chunk, K)
    dst = edge_index[1].reshape(NS, n_chunk, K)
    bn = 1000

    x2 = x.reshape(2 * n, HALF)
    agg1 = _make_seg_sum(n, e, 2, 1)(x2, src, dst)
    t_split = _tc_layer1(agg1, x, W1_rel, W1_root, b1, bn)

    t2 = t_split.reshape(2 * n, HALF)
    agg2 = _make_seg_sum(n, e, 1, n)(t2, src, dst)
    return _tc_layer2(agg2, t_split, W2_rel, W2_root, b2, bn)
